# baseline (speedup 1.0000x reference)
import functools

import jax
import jax.numpy as jnp
from jax import lax
from jax.experimental import pallas as pl
from jax.experimental.pallas import tpu as pltpu
from jax.experimental.pallas import tpu_sc as plsc

_SEQ = 2048
_DIM = 64
_MAXLEN = 160
_NW = 32
_G = 2 * _SEQ
_GROWS = _G // _NW
_ROWS = _SEQ // _NW
_W = 512
_SPAN = _W + _ROWS - 1
_SPAN_PAD = _SPAN + 1

_mesh = plsc.VectorSubcoreMesh(core_axis_name="c", subcore_axis_name="s")


def _build_g_body(pe_hbm, g_hbm, pe_v, g_v):
    wid = lax.axis_index("s") * 2 + lax.axis_index("c")
    base = wid * _GROWS
    pltpu.sync_copy(pe_hbm, pe_v)

    def fill(r, _):
        u = base + r
        idx = jnp.minimum(jnp.maximum(2047 - u, -_MAXLEN), _MAXLEN - 1) + _MAXLEN
        for c in range(_DIM // 16):
            dst = pl.multiple_of(r * _DIM + c * 16, 16)
            g_v[pl.ds(dst, 16)] = pe_v[idx, pl.ds(c * 16, 16)]
        return 0

    lax.fori_loop(0, _GROWS, fill, 0)
    pltpu.sync_copy(g_v, g_hbm.at[pl.ds(base * _DIM, _GROWS * _DIM)])


_build_g = functools.partial(
    pl.kernel,
    out_type=jax.ShapeDtypeStruct((_G * _DIM,), jnp.float32),
    mesh=_mesh,
    scratch_types=[
        pltpu.VMEM((2 * _MAXLEN, _DIM), jnp.float32),
        pltpu.VMEM((_GROWS * _DIM,), jnp.float32),
    ],
)(_build_g_body)


def _emit_body(g_hbm, out_hbm, span_v, buf0, buf1, sem0, sem1):
    wid = lax.axis_index("s") * 2 + lax.axis_index("c")
    r0 = wid * _ROWS
    lanes = lax.broadcasted_iota(jnp.int32, (16,), 0)

    for h in range(_SEQ // _W):
        c0 = h * _W
        s0 = 2047 - (r0 + _ROWS - 1) + c0
        pltpu.sync_copy(
            g_hbm.at[pl.ds(pl.multiple_of(s0 * _DIM, 8), _SPAN_PAD * _DIM)],
            span_v,
        )

        def row(li, buf, sem):
            off = (_ROWS - 1) - li

            def fill(c, _):
                cc = pl.multiple_of(c * 16, 16)
                flat_base = (off + cc + lanes) * _DIM
                for d in range(_DIM):
                    buf[d, pl.ds(cc, 16)] = plsc.load_gather(
                        span_v, [flat_base + d]
                    )
                return 0

            lax.fori_loop(0, _W // 16, fill, 0)
            pltpu.make_async_copy(
                buf, out_hbm.at[r0 + li, :, pl.ds(c0, _W)], sem
            ).start()

        def step(li, _):
            @pl.when(lax.rem(li, 2) == 0)
            def _():
                @pl.when(li >= 2)
                def _():
                    pltpu.make_async_copy(
                        buf0, out_hbm.at[r0, :, pl.ds(c0, _W)], sem0
                    ).wait()

                row(li, buf0, sem0)

            @pl.when(lax.rem(li, 2) == 1)
            def _():
                @pl.when(li >= 2)
                def _():
                    pltpu.make_async_copy(
                        buf1, out_hbm.at[r0, :, pl.ds(c0, _W)], sem1
                    ).wait()

                row(li, buf1, sem1)

            return 0

        lax.fori_loop(0, _ROWS, step, 0)

        pltpu.make_async_copy(buf0, out_hbm.at[r0, :, pl.ds(c0, _W)], sem0).wait()
        pltpu.make_async_copy(buf1, out_hbm.at[r0, :, pl.ds(c0, _W)], sem1).wait()


_emit = functools.partial(
    pl.kernel,
    out_type=jax.ShapeDtypeStruct((_SEQ, _DIM, _SEQ), jnp.float32),
    mesh=_mesh,
    scratch_types=[
        pltpu.VMEM((_SPAN_PAD * _DIM,), jnp.float32),
        pltpu.VMEM((_DIM, _W), jnp.float32),
        pltpu.VMEM((_DIM, _W), jnp.float32),
        pltpu.SemaphoreType.DMA,
        pltpu.SemaphoreType.DMA,
    ],
    compiler_params=pltpu.CompilerParams(needs_layout_passes=False),
)(_emit_body)


@jax.jit
def kernel(hidden_states, pe_k):
    del hidden_states
    g = _build_g(pe_k)
    out = _emit(g)
    return out.transpose(0, 2, 1)


# speedup vs baseline: 2.9574x; 2.9574x over previous
import functools

import jax
import jax.numpy as jnp
from jax import lax
from jax.experimental import pallas as pl
from jax.experimental.pallas import tpu as pltpu
from jax.experimental.pallas import tpu_sc as plsc

_SEQ = 2048
_DIM = 64
_MAXLEN = 160
_NW = 32
_G = 2 * _SEQ
_AD = _DIM // _NW
_ROWS = _SEQ // _NW
_W = 512
_SPAN_LD = 640

_mesh = plsc.VectorSubcoreMesh(core_axis_name="c", subcore_axis_name="s")


def _build_gt_body(pe_hbm, gt_hbm, pe_v, row_v):
    wid = lax.axis_index("s") * 2 + lax.axis_index("c")
    lanes = lax.broadcasted_iota(jnp.int32, (16,), 0)
    pltpu.sync_copy(pe_hbm, pe_v)

    for dl in range(_AD):
        d = wid * _AD + dl
        col_idx = jnp.full((16,), 0, jnp.int32) + d

        def fill(c, _):
            cc = pl.multiple_of(c * 16, 16)
            vec = (2047 - cc) - lanes
            row_idx = (
                jnp.minimum(jnp.maximum(vec, -_MAXLEN), _MAXLEN - 1) + _MAXLEN
            )
            row_v[pl.ds(cc, 16)] = plsc.load_gather(pe_v, [row_idx, col_idx])
            return 0

        lax.fori_loop(0, _G // 16, fill, 0)
        pltpu.sync_copy(row_v, gt_hbm.at[d])


_build_gt = functools.partial(
    pl.kernel,
    out_type=jax.ShapeDtypeStruct((_DIM, _G), jnp.float32),
    mesh=_mesh,
    scratch_types=[
        pltpu.VMEM((2 * _MAXLEN, _DIM), jnp.float32),
        pltpu.VMEM((_G,), jnp.float32),
    ],
    compiler_params=pltpu.CompilerParams(needs_layout_passes=False),
)(_build_gt_body)


def _emit_body(gt_hbm, out_hbm, span_v, buf0, buf1, sem0, sem1):
    wid = lax.axis_index("s") * 2 + lax.axis_index("c")
    r0 = wid * _ROWS
    lanes = lax.broadcasted_iota(jnp.int32, (16,), 0)

    for h in range(_SEQ // _W):
        c0 = h * _W
        s0 = 2047 - (r0 + _ROWS - 1) + c0
        rem = lax.rem(s0, 128)
        sa = pl.multiple_of(s0 - rem, 128)
        pltpu.sync_copy(gt_hbm.at[:, pl.ds(sa, _SPAN_LD)], span_v)

        def row(li, buf, sem):
            off = rem + (_ROWS - 1) - li

            def fill(c, _):
                cc = pl.multiple_of(c * 16, 16)
                idx = off + cc + lanes
                for d in range(_DIM):
                    dvec = jnp.full((16,), d, jnp.int32)
                    buf[d, pl.ds(cc, 16)] = plsc.load_gather(
                        span_v, [dvec, idx]
                    )
                return 0

            lax.fori_loop(0, _W // 16, fill, 0)
            pltpu.make_async_copy(
                buf, out_hbm.at[r0 + li, :, pl.ds(c0, _W)], sem
            ).start()

        def step(li, _):
            @pl.when(lax.rem(li, 2) == 0)
            def _():
                @pl.when(li >= 2)
                def _():
                    pltpu.make_async_copy(
                        buf0, out_hbm.at[r0, :, pl.ds(c0, _W)], sem0
                    ).wait()

                row(li, buf0, sem0)

            @pl.when(lax.rem(li, 2) == 1)
            def _():
                @pl.when(li >= 2)
                def _():
                    pltpu.make_async_copy(
                        buf1, out_hbm.at[r0, :, pl.ds(c0, _W)], sem1
                    ).wait()

                row(li, buf1, sem1)

            return 0

        lax.fori_loop(0, _ROWS, step, 0)

        pltpu.make_async_copy(buf0, out_hbm.at[r0, :, pl.ds(c0, _W)], sem0).wait()
        pltpu.make_async_copy(buf1, out_hbm.at[r0, :, pl.ds(c0, _W)], sem1).wait()


_emit = functools.partial(
    pl.kernel,
    out_type=jax.ShapeDtypeStruct((_SEQ, _DIM, _SEQ), jnp.float32),
    mesh=_mesh,
    scratch_types=[
        pltpu.VMEM((_DIM, _SPAN_LD), jnp.float32),
        pltpu.VMEM((_DIM, _W), jnp.float32),
        pltpu.VMEM((_DIM, _W), jnp.float32),
        pltpu.SemaphoreType.DMA,
        pltpu.SemaphoreType.DMA,
    ],
    compiler_params=pltpu.CompilerParams(needs_layout_passes=False),
)(_emit_body)


@jax.jit
def kernel(hidden_states, pe_k):
    del hidden_states
    gt = _build_gt(pe_k)
    out = _emit(gt)
    return out.transpose(0, 2, 1)


# speedup vs baseline: 5.0672x; 1.7134x over previous
import functools

import jax
import jax.numpy as jnp
from jax import lax
from jax.experimental import pallas as pl
from jax.experimental.pallas import tpu as pltpu
from jax.experimental.pallas import tpu_sc as plsc

_SEQ = 2048
_DIM = 64
_MAXLEN = 160
_NW = 32
_G = 4224
_AD = _DIM // _NW
_ROWS = _SEQ // _NW
_W = 512
_SPAN_LD = 768

_mesh = plsc.VectorSubcoreMesh(core_axis_name="c", subcore_axis_name="s")


def _build_gt_body(pe_hbm, gt_hbm, pe_v, row_v):
    wid = lax.axis_index("s") * 2 + lax.axis_index("c")
    lanes = lax.broadcasted_iota(jnp.int32, (16,), 0)
    pltpu.sync_copy(pe_hbm, pe_v)

    for dl in range(_AD):
        d = wid * _AD + dl
        col_idx = jnp.full((16,), 0, jnp.int32) + d

        def fill(c, _):
            cc = pl.multiple_of(c * 16, 16)
            vec = (2047 - cc) - lanes
            row_idx = (
                jnp.minimum(jnp.maximum(vec, -_MAXLEN), _MAXLEN - 1) + _MAXLEN
            )
            row_v[pl.ds(cc, 16)] = plsc.load_gather(pe_v, [row_idx, col_idx])
            return 0

        lax.fori_loop(0, _G // 16, fill, 0)
        pltpu.sync_copy(row_v, gt_hbm.at[d])


_build_gt = functools.partial(
    pl.kernel,
    out_type=jax.ShapeDtypeStruct((_DIM, _G), jnp.float32),
    mesh=_mesh,
    scratch_types=[
        pltpu.VMEM((2 * _MAXLEN, _DIM), jnp.float32),
        pltpu.VMEM((_G,), jnp.float32),
    ],
    compiler_params=pltpu.CompilerParams(needs_layout_passes=False),
)(_build_gt_body)


def _emit_body(gt_hbm, out_hbm, span_v, buf0, buf1, sem0, sem1, semc):
    wid = lax.axis_index("s") * 2 + lax.axis_index("c")
    r0 = wid * _ROWS
    lanes = lax.broadcasted_iota(jnp.int32, (16,), 0)

    for h in range(_SEQ // _W):
        c0 = h * _W
        s0 = 2047 - (r0 + _ROWS - 1) + c0
        rem = lax.rem(s0, 128)
        sa = pl.multiple_of(s0 - rem, 128)
        pltpu.sync_copy(gt_hbm.at[:, pl.ds(sa, _SPAN_LD)], span_v)

        count0 = jnp.clip(s0 - 2143, 0, _ROWS)
        hi = jnp.clip(s0 - 1314, 0, _ROWS)

        def const_row(woff):
            def body(li, _):
                pltpu.make_async_copy(
                    span_v.at[:, pl.ds(woff, _W)],
                    out_hbm.at[r0 + li, :, pl.ds(c0, _W)],
                    semc,
                ).start()
                return 0

            return body

        lax.fori_loop(0, count0, const_row(256), 0)
        lax.fori_loop(hi, _ROWS, const_row(0), 0)

        def row(li, buf, sem):
            off = rem + (_ROWS - 1) - li

            def fill(c, _):
                cc = pl.multiple_of(c * 16, 16)
                idx = off + cc + lanes
                for d in range(_DIM):
                    dvec = jnp.full((16,), d, jnp.int32)
                    buf[d, pl.ds(cc, 16)] = plsc.load_gather(
                        span_v, [dvec, idx]
                    )
                return 0

            lax.fori_loop(0, _W // 16, fill, 0)
            pltpu.make_async_copy(
                buf, out_hbm.at[r0 + li, :, pl.ds(c0, _W)], sem
            ).start()

        def step(li, _):
            vi = li - count0

            @pl.when(lax.rem(vi, 2) == 0)
            def _():
                @pl.when(vi >= 2)
                def _():
                    pltpu.make_async_copy(
                        buf0, out_hbm.at[r0, :, pl.ds(c0, _W)], sem0
                    ).wait()

                row(li, buf0, sem0)

            @pl.when(lax.rem(vi, 2) == 1)
            def _():
                @pl.when(vi >= 2)
                def _():
                    pltpu.make_async_copy(
                        buf1, out_hbm.at[r0, :, pl.ds(c0, _W)], sem1
                    ).wait()

                row(li, buf1, sem1)

            return 0

        lax.fori_loop(count0, hi, step, 0)

        nvar = hi - count0

        def drainc(i, _):
            pltpu.make_async_copy(
                span_v.at[:, pl.ds(0, _W)],
                out_hbm.at[r0, :, pl.ds(c0, _W)],
                semc,
            ).wait()
            return 0

        lax.fori_loop(0, count0 + (_ROWS - hi), drainc, 0)

        @pl.when(nvar >= 1)
        def _():
            pltpu.make_async_copy(
                buf0, out_hbm.at[r0, :, pl.ds(c0, _W)], sem0
            ).wait()

        @pl.when(nvar >= 2)
        def _():
            pltpu.make_async_copy(
                buf1, out_hbm.at[r0, :, pl.ds(c0, _W)], sem1
            ).wait()


_emit = functools.partial(
    pl.kernel,
    out_type=jax.ShapeDtypeStruct((_SEQ, _DIM, _SEQ), jnp.float32),
    mesh=_mesh,
    scratch_types=[
        pltpu.VMEM((_DIM, _SPAN_LD), jnp.float32),
        pltpu.VMEM((_DIM, _W), jnp.float32),
        pltpu.VMEM((_DIM, _W), jnp.float32),
        pltpu.SemaphoreType.DMA,
        pltpu.SemaphoreType.DMA,
        pltpu.SemaphoreType.DMA,
    ],
    compiler_params=pltpu.CompilerParams(needs_layout_passes=False),
)(_emit_body)


@jax.jit
def kernel(hidden_states, pe_k):
    del hidden_states
    gt = _build_gt(pe_k)
    out = _emit(gt)
    return out.transpose(0, 2, 1)


# speedup vs baseline: 7.0519x; 1.3917x over previous
import functools

import jax
import jax.numpy as jnp
from jax import lax
from jax.experimental import pallas as pl
from jax.experimental.pallas import tpu as pltpu
from jax.experimental.pallas import tpu_sc as plsc

_SEQ = 2048
_DIM = 64
_MAXLEN = 160
_NW = 32
_G = 4224
_AD = _DIM // _NW
_ROWS = _SEQ // _NW
_W = 256
_SPAN_LD = 512

_mesh = plsc.VectorSubcoreMesh(core_axis_name="c", subcore_axis_name="s")


def _build_gt_body(pe_hbm, gt_hbm, pe_v, row_v):
    wid = lax.axis_index("s") * 2 + lax.axis_index("c")
    lanes = lax.broadcasted_iota(jnp.int32, (16,), 0)
    pltpu.sync_copy(pe_hbm, pe_v)

    for dl in range(_AD):
        d = wid * _AD + dl
        col_idx = jnp.full((16,), 0, jnp.int32) + d

        def fill(c, _):
            cc = pl.multiple_of(c * 16, 16)
            vec = (2047 - cc) - lanes
            row_idx = (
                jnp.minimum(jnp.maximum(vec, -_MAXLEN), _MAXLEN - 1) + _MAXLEN
            )
            row_v[pl.ds(cc, 16)] = plsc.load_gather(pe_v, [row_idx, col_idx])
            return 0

        lax.fori_loop(0, _G // 16, fill, 0)
        pltpu.sync_copy(row_v, gt_hbm.at[d])


_build_gt = functools.partial(
    pl.kernel,
    out_type=jax.ShapeDtypeStruct((_DIM, _G), jnp.float32),
    mesh=_mesh,
    scratch_types=[
        pltpu.VMEM((2 * _MAXLEN, _DIM), jnp.float32),
        pltpu.VMEM((_G,), jnp.float32),
    ],
    compiler_params=pltpu.CompilerParams(needs_layout_passes=False),
)(_build_gt_body)


def _emit_body(gt_hbm, out_hbm, span_v, buf0, buf1, cb0, cb319, sem0, sem1, semc):
    wid = lax.axis_index("s") * 2 + lax.axis_index("c")
    r0 = wid * _ROWS
    lanes = lax.broadcasted_iota(jnp.int32, (16,), 0)

    pltpu.sync_copy(gt_hbm.at[:, pl.ds(0, _W)], cb319)
    pltpu.sync_copy(gt_hbm.at[:, pl.ds(2304, _W)], cb0)

    def task(h, nconst):
        c0 = pl.multiple_of(h * _W, 128)
        s0 = 2047 - (r0 + _ROWS - 1) + c0
        rem = lax.rem(s0, 128)
        sa = pl.multiple_of(s0 - rem, 128)

        count0 = jnp.clip(s0 - 2143, 0, _ROWS)
        hi = jnp.clip(s0 - 1570, 0, _ROWS)

        pltpu.sync_copy(gt_hbm.at[:, pl.ds(sa, _SPAN_LD)], span_v)

        def row(li, buf, sem):
            off = rem + (_ROWS - 1) - li

            def fill(c, _):
                cc = pl.multiple_of(c * 16, 16)
                idx = off + cc + lanes
                for d in range(_DIM):
                    dvec = jnp.full((16,), d, jnp.int32)
                    buf[d, pl.ds(cc, 16)] = plsc.load_gather(
                        span_v, [dvec, idx]
                    )
                return 0

            lax.fori_loop(0, _W // 16, fill, 0)
            pltpu.make_async_copy(
                buf, out_hbm.at[r0 + li, :, pl.ds(c0, _W)], sem
            ).start()

        def step(li, _):
            vi = li - count0

            @pl.when(lax.rem(vi, 2) == 0)
            def _():
                @pl.when(vi >= 2)
                def _():
                    pltpu.make_async_copy(
                        buf0, out_hbm.at[r0, :, pl.ds(c0, _W)], sem0
                    ).wait()

                row(li, buf0, sem0)

            @pl.when(lax.rem(vi, 2) == 1)
            def _():
                @pl.when(vi >= 2)
                def _():
                    pltpu.make_async_copy(
                        buf1, out_hbm.at[r0, :, pl.ds(c0, _W)], sem1
                    ).wait()

                row(li, buf1, sem1)

            return 0

        lax.fori_loop(count0, hi, step, 0)

        def const_row(cb):
            def body(li, _):
                pltpu.make_async_copy(
                    cb, out_hbm.at[r0 + li, :, pl.ds(c0, _W)], semc
                ).start()
                return 0

            return body

        lax.fori_loop(0, count0, const_row(cb0), 0)
        lax.fori_loop(hi, _ROWS, const_row(cb319), 0)
        nvar = hi - count0

        @pl.when(nvar >= 1)
        def _():
            pltpu.make_async_copy(
                buf0, out_hbm.at[r0, :, pl.ds(c0, _W)], sem0
            ).wait()

        @pl.when(nvar >= 2)
        def _():
            pltpu.make_async_copy(
                buf1, out_hbm.at[r0, :, pl.ds(c0, _W)], sem1
            ).wait()

        return nconst + count0 + (_ROWS - hi)

    nconst = lax.fori_loop(0, _SEQ // _W, task, 0)

    def drainc(i, _):
        pltpu.make_async_copy(
            cb0, out_hbm.at[r0, :, pl.ds(0, _W)], semc
        ).wait()
        return 0

    lax.fori_loop(0, nconst, drainc, 0)


_emit = functools.partial(
    pl.kernel,
    out_type=jax.ShapeDtypeStruct((_SEQ, _DIM, _SEQ), jnp.float32),
    mesh=_mesh,
    scratch_types=[
        pltpu.VMEM((_DIM, _SPAN_LD), jnp.float32),
        pltpu.VMEM((_DIM, _W), jnp.float32),
        pltpu.VMEM((_DIM, _W), jnp.float32),
        pltpu.VMEM((_DIM, _W), jnp.float32),
        pltpu.VMEM((_DIM, _W), jnp.float32),
        pltpu.SemaphoreType.DMA,
        pltpu.SemaphoreType.DMA,
        pltpu.SemaphoreType.DMA,
    ],
    compiler_params=pltpu.CompilerParams(needs_layout_passes=False),
)(_emit_body)


@jax.jit
def kernel(hidden_states, pe_k):
    del hidden_states
    gt = _build_gt(pe_k)
    out = _emit(gt)
    return out.transpose(0, 2, 1)
